# baseline (speedup 1.0000x reference)
import functools

import jax
import jax.numpy as jnp
from jax import lax
from jax.experimental import pallas as pl
from jax.experimental.pallas import tpu as pltpu
from jax.experimental.pallas import tpu_sc as plsc

NC = 2
NS = 16
NW = NC * NS

CW = 128
DEPTH = 4

_CPARAMS = pltpu.CompilerParams(
    use_tc_tiling_on_sc=True, needs_layout_passes=False)


def _make_relayout(V, D):
    NT = V // CW
    VTAIL = V - NT * CW
    P2 = CW // 2
    mesh = plsc.VectorSubcoreMesh(core_axis_name="c", subcore_axis_name="s")

    @functools.partial(
        pl.kernel,
        out_type=jax.ShapeDtypeStruct((V // 2, 2 * D), jnp.float32),
        mesh=mesh,
        scratch_types=(
            [pltpu.VMEM((D, CW), jnp.float32) for _ in range(2)]
            + [pltpu.VMEM((P2, 2 * D), jnp.float32) for _ in range(2)]
            + [pltpu.SemaphoreType.DMA for _ in range(4)]
        ),
        compiler_params=_CPARAMS,
    )
    def k(tt_hbm, tail_hbm, out_hbm, slab0, slab1, tb0, tb1, s0, s1, w0, w1):
        wid = lax.axis_index("s") * NC + lax.axis_index("c")
        lo = wid * NT // NW
        hi = (wid + 1) * NT // NW
        iota16 = lax.iota(jnp.int32, 16)
        slabs = (slab0, slab1)
        tbs = (tb0, tb1)
        ssems = (s0, s1)
        wsems = (w0, w1)

        def slab_copy(vt, p):
            return pltpu.make_async_copy(
                tt_hbm.at[:, pl.ds(vt * CW, CW)], slabs[p], ssems[p])

        def write_copy(vt, p):
            return pltpu.make_async_copy(
                tbs[p], out_hbm.at[pl.ds(vt * P2, P2)], wsems[p])

        def transpose(p):
            @pl.loop(0, P2)
            def _p(pr):
                c0 = iota16 * 0 + 2 * pr
                c1 = c0 + 1
                vals = []
                for g in range(8):
                    rid = (g % 4) * 16 + iota16
                    vals.append(plsc.load_gather(
                        slabs[p], [rid, c0 if g < 4 else c1]))
                for g in range(8):
                    tbs[p][pr, pl.ds(g * 16, 16)] = vals[g]

        slab_copy(lo, 0).start()

        @pl.loop(0, 1)
        def _main(_):
            @pl.loop(lo, hi)
            def _vt(vt):
                t = vt - lo
                p = jnp.mod(t, 2)

                def do(pp):
                    slab_copy(vt, pp).wait()
                    @pl.when(vt + 1 < hi)
                    def _n():
                        slab_copy(vt + 1, 1 - pp).start()
                    @pl.when(t >= 2)
                    def _dw():
                        write_copy(vt - 2, pp).wait()
                    transpose(pp)
                    write_copy(vt, pp).start()

                @pl.when(p == 0)
                def _e():
                    do(0)

                @pl.when(p == 1)
                def _o():
                    do(1)

            par_last = jnp.mod(hi - 1 - lo, 2)
            for pp in range(2):
                vt_pp = jnp.where(par_last == pp, hi - 1, hi - 2)

                @pl.when(vt_pp >= lo)
                def _dw(vt_pp=vt_pp, pp=pp):
                    write_copy(vt_pp, pp).wait()

        @pl.when(wid == 0)
        def _tail():
            PT = VTAIL // 2
            pltpu.make_async_copy(tail_hbm, slab0, s0).start()
            pltpu.make_async_copy(tail_hbm, slab0, s0).wait()
            transpose(0)
            pltpu.make_async_copy(
                tb0.at[pl.ds(P2 - PT, PT)],
                out_hbm.at[pl.ds(NT * P2, PT)], w0).start()
            pltpu.make_async_copy(
                tb0.at[pl.ds(P2 - PT, PT)],
                out_hbm.at[pl.ds(NT * P2, PT)], w0).wait()

    return k


def _make_gather(V2, D, NJ, NI):
    nq = NJ * (NI // CW)
    q_per_w = nq // NW
    ic_per_j = NI // CW
    DT, DD = D // 8, 8
    mesh = plsc.VectorSubcoreMesh(core_axis_name="c", subcore_axis_name="s")

    @functools.partial(
        pl.kernel,
        out_type=jax.ShapeDtypeStruct((NJ, DT, ic_per_j, DD, CW), jnp.float32),
        mesh=mesh,
        scratch_types=(
            [pltpu.VMEM((CW,), jnp.int32) for _ in range(DEPTH)]
            + [pltpu.VMEM((CW,), jnp.int32) for _ in range(DEPTH)]
            + [pltpu.VMEM((CW, 2 * D), jnp.float32) for _ in range(DEPTH)]
            + [pltpu.VMEM((D, CW), jnp.float32) for _ in range(DEPTH)]
            + [pltpu.SemaphoreType.DMA for _ in range(3 * DEPTH)]
        ),
        compiler_params=_CPARAMS,
    )
    def k(table_hbm, idx_hbm, out_hbm, *bufs):
        idxb = bufs[:DEPTH]
        idxh = bufs[DEPTH:2 * DEPTH]
        rows = bufs[2 * DEPTH:3 * DEPTH]
        tbuf = bufs[3 * DEPTH:4 * DEPTH]
        isem = bufs[4 * DEPTH:5 * DEPTH]
        gsem = bufs[5 * DEPTH:6 * DEPTH]
        wsem = bufs[6 * DEPTH:]
        wid = lax.axis_index("s") * NC + lax.axis_index("c")
        q0 = wid * q_per_w
        iota16 = lax.iota(jnp.int32, 16)

        def idx_copy(q, s):
            return pltpu.make_async_copy(
                idx_hbm.at[pl.ds(q * CW, CW)], idxb[s], isem[s])

        def gather_copy(s):
            return pltpu.make_async_copy(table_hbm.at[idxh[s]], rows[s], gsem[s])

        def start_gather(s):
            for g in range(CW // 16):
                idxh[s][pl.ds(g * 16, 16)] = jax.lax.shift_right_logical(
                    idxb[s][pl.ds(g * 16, 16)], 1)
            gather_copy(s).start()

        def write_copies(q, s):
            j = q // ic_per_j
            it = q % ic_per_j
            return [
                pltpu.make_async_copy(
                    tbuf[s].at[pl.ds(dt * DD, DD)], out_hbm.at[j, dt, it],
                    wsem[s])
                for dt in range(DT)
            ]

        def transpose(s):
            for g in range(CW // 16):
                rid = g * 16 + iota16
                par = (idxb[s][pl.ds(g * 16, 16)] & 1) * D

                @pl.loop(0, D, step=4)
                def _d(d):
                    v0 = plsc.load_gather(rows[s], [rid, par + d])
                    v1 = plsc.load_gather(rows[s], [rid, par + (d + 1)])
                    v2 = plsc.load_gather(rows[s], [rid, par + (d + 2)])
                    v3 = plsc.load_gather(rows[s], [rid, par + (d + 3)])
                    tbuf[s][d, pl.ds(g * 16, 16)] = v0
                    tbuf[s][d + 1, pl.ds(g * 16, 16)] = v1
                    tbuf[s][d + 2, pl.ds(g * 16, 16)] = v2
                    tbuf[s][d + 3, pl.ds(g * 16, 16)] = v3

        for b in range(DEPTH):
            idx_copy(q0 + b, b).start()
        for b in range(2):
            idx_copy(q0 + b, b).wait()
            start_gather(b)

        @pl.loop(0, q_per_w, step=DEPTH)
        def _group(t):
            for b in range(DEPTH):
                tq = t + b
                q = q0 + tq
                s = b
                gather_copy(s).wait()
                @pl.when(tq >= DEPTH)
                def _drainw():
                    for c in write_copies(q - DEPTH, s):
                        c.wait()
                transpose(s)
                @pl.when(tq + DEPTH < q_per_w)
                def _nexti():
                    idx_copy(q + DEPTH, s).start()
                for c in write_copies(q, s):
                    c.start()
                s2 = (b + 2) % DEPTH
                @pl.when(tq + 2 < q_per_w)
                def _nextg():
                    idx_copy(q + 2, s2).wait()
                    start_gather(s2)

        for b in range(DEPTH):
            tq = q_per_w - DEPTH + b
            for c in write_copies(q0 + tq, tq % DEPTH):
                c.wait()

    return k


@jax.jit
def kernel(sparse_table, indices):
    n0, n1 = indices.shape
    V, D = sparse_table.shape
    tt = sparse_table.T
    tail = jax.lax.slice(tt, (0, V - 128), (D, V))
    table2 = _make_relayout(V, D)(tt, tail)
    idx_t = indices.T.astype(jnp.int32).reshape(-1)
    out6 = _make_gather(V // 2, D, n1, n0)(table2, idx_t)
    return out6.transpose(2, 4, 0, 1, 3).reshape(n0, n1, D)


# speedup vs baseline: 1.0133x; 1.0133x over previous
import functools

import jax
import jax.numpy as jnp
from jax import lax
from jax.experimental import pallas as pl
from jax.experimental.pallas import tpu as pltpu
from jax.experimental.pallas import tpu_sc as plsc

NC = 2
NS = 16
NW = NC * NS

CW = 128
DEPTH = 4

_CPARAMS = pltpu.CompilerParams(
    use_tc_tiling_on_sc=True, needs_layout_passes=False,
    disable_bounds_checks=True)


def _make_relayout(V, D):
    NT = V // CW
    VTAIL = V - NT * CW
    P2 = CW // 2
    mesh = plsc.VectorSubcoreMesh(core_axis_name="c", subcore_axis_name="s")

    @functools.partial(
        pl.kernel,
        out_type=jax.ShapeDtypeStruct((V // 2, 2 * D), jnp.float32),
        mesh=mesh,
        scratch_types=(
            [pltpu.VMEM((D, CW), jnp.float32) for _ in range(2)]
            + [pltpu.VMEM((P2, 2 * D), jnp.float32) for _ in range(2)]
            + [pltpu.SemaphoreType.DMA for _ in range(4)]
        ),
        compiler_params=_CPARAMS,
    )
    def k(tt_hbm, tail_hbm, out_hbm, slab0, slab1, tb0, tb1, s0, s1, w0, w1):
        wid = lax.axis_index("s") * NC + lax.axis_index("c")
        lo = wid * NT // NW
        hi = (wid + 1) * NT // NW
        iota16 = lax.iota(jnp.int32, 16)
        slabs = (slab0, slab1)
        tbs = (tb0, tb1)
        ssems = (s0, s1)
        wsems = (w0, w1)

        def slab_copy(vt, p):
            return pltpu.make_async_copy(
                tt_hbm.at[:, pl.ds(vt * CW, CW)], slabs[p], ssems[p])

        def write_copy(vt, p):
            return pltpu.make_async_copy(
                tbs[p], out_hbm.at[pl.ds(vt * P2, P2)], wsems[p])

        def transpose(p):
            @pl.loop(0, P2, step=2)
            def _p(pr):
                for u in range(2):
                    c0 = iota16 * 0 + 2 * (pr + u)
                    c1 = c0 + 1
                    vals = []
                    for g in range(8):
                        rid = (g % 4) * 16 + iota16
                        vals.append(plsc.load_gather(
                            slabs[p], [rid, c0 if g < 4 else c1]))
                    for g in range(8):
                        tbs[p][pr + u, pl.ds(g * 16, 16)] = vals[g]

        slab_copy(lo, 0).start()

        @pl.loop(0, 1)
        def _main(_):
            @pl.loop(lo, hi)
            def _vt(vt):
                t = vt - lo
                p = jnp.mod(t, 2)

                def do(pp):
                    slab_copy(vt, pp).wait()
                    @pl.when(vt + 1 < hi)
                    def _n():
                        slab_copy(vt + 1, 1 - pp).start()
                    @pl.when(t >= 2)
                    def _dw():
                        write_copy(vt - 2, pp).wait()
                    transpose(pp)
                    write_copy(vt, pp).start()

                @pl.when(p == 0)
                def _e():
                    do(0)

                @pl.when(p == 1)
                def _o():
                    do(1)

            par_last = jnp.mod(hi - 1 - lo, 2)
            for pp in range(2):
                vt_pp = jnp.where(par_last == pp, hi - 1, hi - 2)

                @pl.when(vt_pp >= lo)
                def _dw(vt_pp=vt_pp, pp=pp):
                    write_copy(vt_pp, pp).wait()

        @pl.when(wid == 0)
        def _tail():
            PT = VTAIL // 2
            pltpu.make_async_copy(tail_hbm, slab0, s0).start()
            pltpu.make_async_copy(tail_hbm, slab0, s0).wait()
            transpose(0)
            pltpu.make_async_copy(
                tb0.at[pl.ds(P2 - PT, PT)],
                out_hbm.at[pl.ds(NT * P2, PT)], w0).start()
            pltpu.make_async_copy(
                tb0.at[pl.ds(P2 - PT, PT)],
                out_hbm.at[pl.ds(NT * P2, PT)], w0).wait()

    return k


def _make_gather(V2, D, NJ, NI):
    nq = NJ * (NI // CW)
    q_per_w = nq // NW
    ic_per_j = NI // CW
    DT, DD = D // 8, 8
    mesh = plsc.VectorSubcoreMesh(core_axis_name="c", subcore_axis_name="s")

    @functools.partial(
        pl.kernel,
        out_type=jax.ShapeDtypeStruct((NJ, DT, ic_per_j, DD, CW), jnp.float32),
        mesh=mesh,
        scratch_types=(
            [pltpu.VMEM((CW,), jnp.int32) for _ in range(DEPTH)]
            + [pltpu.VMEM((CW,), jnp.int32) for _ in range(DEPTH)]
            + [pltpu.VMEM((CW, 2 * D), jnp.float32) for _ in range(DEPTH)]
            + [pltpu.VMEM((D, CW), jnp.float32) for _ in range(DEPTH)]
            + [pltpu.SemaphoreType.DMA for _ in range(3 * DEPTH)]
        ),
        compiler_params=_CPARAMS,
    )
    def k(table_hbm, idx_hbm, out_hbm, *bufs):
        idxb = bufs[:DEPTH]
        idxh = bufs[DEPTH:2 * DEPTH]
        rows = bufs[2 * DEPTH:3 * DEPTH]
        tbuf = bufs[3 * DEPTH:4 * DEPTH]
        isem = bufs[4 * DEPTH:5 * DEPTH]
        gsem = bufs[5 * DEPTH:6 * DEPTH]
        wsem = bufs[6 * DEPTH:]
        wid = lax.axis_index("s") * NC + lax.axis_index("c")
        q0 = wid * q_per_w
        iota16 = lax.iota(jnp.int32, 16)

        def idx_copy(q, s):
            return pltpu.make_async_copy(
                idx_hbm.at[pl.ds(q * CW, CW)], idxb[s], isem[s])

        def gather_copy(s):
            return pltpu.make_async_copy(table_hbm.at[idxh[s]], rows[s], gsem[s])

        def start_gather(s):
            for g in range(CW // 16):
                idxh[s][pl.ds(g * 16, 16)] = jax.lax.shift_right_logical(
                    idxb[s][pl.ds(g * 16, 16)], 1)
            gather_copy(s).start()

        def write_copies(q, s):
            j = q // ic_per_j
            it = q % ic_per_j
            return [
                pltpu.make_async_copy(
                    tbuf[s].at[pl.ds(dt * DD, DD)], out_hbm.at[j, dt, it],
                    wsem[s])
                for dt in range(DT)
            ]

        def transpose(s):
            for g in range(CW // 16):
                rid = g * 16 + iota16
                par = (idxb[s][pl.ds(g * 16, 16)] & 1) * D

                @pl.loop(0, D, step=4)
                def _d(d):
                    v0 = plsc.load_gather(rows[s], [rid, par + d])
                    v1 = plsc.load_gather(rows[s], [rid, par + (d + 1)])
                    v2 = plsc.load_gather(rows[s], [rid, par + (d + 2)])
                    v3 = plsc.load_gather(rows[s], [rid, par + (d + 3)])
                    tbuf[s][d, pl.ds(g * 16, 16)] = v0
                    tbuf[s][d + 1, pl.ds(g * 16, 16)] = v1
                    tbuf[s][d + 2, pl.ds(g * 16, 16)] = v2
                    tbuf[s][d + 3, pl.ds(g * 16, 16)] = v3

        for b in range(DEPTH):
            idx_copy(q0 + b, b).start()
        for b in range(2):
            idx_copy(q0 + b, b).wait()
            start_gather(b)

        @pl.loop(0, q_per_w, step=DEPTH)
        def _group(t):
            for b in range(DEPTH):
                tq = t + b
                q = q0 + tq
                s = b
                gather_copy(s).wait()
                @pl.when(tq >= DEPTH)
                def _drainw():
                    for c in write_copies(q - DEPTH, s):
                        c.wait()
                transpose(s)
                @pl.when(tq + DEPTH < q_per_w)
                def _nexti():
                    idx_copy(q + DEPTH, s).start()
                for c in write_copies(q, s):
                    c.start()
                s2 = (b + 2) % DEPTH
                @pl.when(tq + 2 < q_per_w)
                def _nextg():
                    idx_copy(q + 2, s2).wait()
                    start_gather(s2)

        for b in range(DEPTH):
            tq = q_per_w - DEPTH + b
            for c in write_copies(q0 + tq, tq % DEPTH):
                c.wait()

    return k


@jax.jit
def kernel(sparse_table, indices):
    n0, n1 = indices.shape
    V, D = sparse_table.shape
    tt = sparse_table.T
    tail = jax.lax.slice(tt, (0, V - 128), (D, V))
    table2 = _make_relayout(V, D)(tt, tail)
    idx_t = indices.T.astype(jnp.int32).reshape(-1)
    out6 = _make_gather(V // 2, D, n1, n0)(table2, idx_t)
    return out6.transpose(2, 4, 0, 1, 3).reshape(n0, n1, D)


# speedup vs baseline: 1.1291x; 1.1142x over previous
import functools

import jax
import jax.numpy as jnp
from jax import lax
from jax.experimental import pallas as pl
from jax.experimental.pallas import tpu as pltpu
from jax.experimental.pallas import tpu_sc as plsc

NC = 2
NS = 16
NW = NC * NS

CW = 128
DEPTH = 4

_CPARAMS = pltpu.CompilerParams(
    use_tc_tiling_on_sc=True, needs_layout_passes=False,
    disable_bounds_checks=True)


def _make_relayout(V, D):
    NT = V // CW
    VTAIL = V - NT * CW
    P2 = CW // 2
    mesh = plsc.VectorSubcoreMesh(core_axis_name="c", subcore_axis_name="s")

    @functools.partial(
        pl.kernel,
        out_type=jax.ShapeDtypeStruct((V // 2, 2 * D), jnp.float32),
        mesh=mesh,
        scratch_types=(
            [pltpu.VMEM((D, CW + 1), jnp.float32) for _ in range(2)]
            + [pltpu.VMEM((P2, 2 * D), jnp.float32) for _ in range(2)]
            + [pltpu.SemaphoreType.DMA for _ in range(4)]
        ),
        compiler_params=_CPARAMS,
    )
    def k(tt_hbm, tail_hbm, out_hbm, slab0, slab1, tb0, tb1, s0, s1, w0, w1):
        wid = lax.axis_index("s") * NC + lax.axis_index("c")
        lo = wid * NT // NW
        hi = (wid + 1) * NT // NW
        iota16 = lax.iota(jnp.int32, 16)
        slabs = (slab0, slab1)
        tbs = (tb0, tb1)
        ssems = (s0, s1)
        wsems = (w0, w1)

        def slab_copy(vt, p):
            return pltpu.make_async_copy(
                tt_hbm.at[:, pl.ds(vt * CW, CW)],
                slabs[p].at[:, pl.ds(0, CW)], ssems[p])

        def write_copy(vt, p):
            return pltpu.make_async_copy(
                tbs[p], out_hbm.at[pl.ds(vt * P2, P2)], wsems[p])

        def transpose(p):
            @pl.loop(0, P2, step=2)
            def _p(pr):
                for u in range(2):
                    c0 = iota16 * 0 + 2 * (pr + u)
                    c1 = c0 + 1
                    vals = []
                    for g in range(8):
                        rid = (g % 4) * 16 + iota16
                        vals.append(plsc.load_gather(
                            slabs[p], [rid, c0 if g < 4 else c1]))
                    for g in range(8):
                        tbs[p][pr + u, pl.ds(g * 16, 16)] = vals[g]

        slab_copy(lo, 0).start()

        @pl.loop(0, 1)
        def _main(_):
            @pl.loop(lo, hi)
            def _vt(vt):
                t = vt - lo
                p = jnp.mod(t, 2)

                def do(pp):
                    slab_copy(vt, pp).wait()
                    @pl.when(vt + 1 < hi)
                    def _n():
                        slab_copy(vt + 1, 1 - pp).start()
                    @pl.when(t >= 2)
                    def _dw():
                        write_copy(vt - 2, pp).wait()
                    transpose(pp)
                    write_copy(vt, pp).start()

                @pl.when(p == 0)
                def _e():
                    do(0)

                @pl.when(p == 1)
                def _o():
                    do(1)

            par_last = jnp.mod(hi - 1 - lo, 2)
            for pp in range(2):
                vt_pp = jnp.where(par_last == pp, hi - 1, hi - 2)

                @pl.when(vt_pp >= lo)
                def _dw(vt_pp=vt_pp, pp=pp):
                    write_copy(vt_pp, pp).wait()

        @pl.when(wid == 0)
        def _tail():
            PT = VTAIL // 2
            pltpu.make_async_copy(
                tail_hbm, slab0.at[:, pl.ds(0, CW)], s0).start()
            pltpu.make_async_copy(
                tail_hbm, slab0.at[:, pl.ds(0, CW)], s0).wait()
            transpose(0)
            pltpu.make_async_copy(
                tb0.at[pl.ds(P2 - PT, PT)],
                out_hbm.at[pl.ds(NT * P2, PT)], w0).start()
            pltpu.make_async_copy(
                tb0.at[pl.ds(P2 - PT, PT)],
                out_hbm.at[pl.ds(NT * P2, PT)], w0).wait()

    return k


def _make_gather(V, D, NJ, NI):
    nq = NJ * (NI // CW)
    q_per_w = nq // NW
    ic_per_j = NI // CW
    DT, DD = D // 8, 8
    mesh = plsc.VectorSubcoreMesh(core_axis_name="c", subcore_axis_name="s")

    @functools.partial(
        pl.kernel,
        out_type=jax.ShapeDtypeStruct((NJ, DT, ic_per_j, DD, CW), jnp.float32),
        mesh=mesh,
        scratch_types=(
            [pltpu.VMEM((CW,), jnp.int32) for _ in range(DEPTH)]
            + [pltpu.VMEM((CW, D), jnp.float32) for _ in range(DEPTH)]
            + [pltpu.VMEM((D, CW + 1), jnp.float32) for _ in range(DEPTH)]
            + [pltpu.SemaphoreType.DMA for _ in range(3 * DEPTH)]
        ),
        compiler_params=pltpu.CompilerParams(
            use_tc_tiling_on_sc=False, needs_layout_passes=False,
            disable_bounds_checks=True),
    )
    def k(table_hbm, idx_hbm, out_hbm, *bufs):
        idxb = bufs[:DEPTH]
        rows = bufs[DEPTH:2 * DEPTH]
        tbuf = bufs[2 * DEPTH:3 * DEPTH]
        isem = bufs[3 * DEPTH:4 * DEPTH]
        gsem = bufs[4 * DEPTH:5 * DEPTH]
        wsem = bufs[5 * DEPTH:]
        wid = lax.axis_index("s") * NC + lax.axis_index("c")
        q0 = wid * q_per_w
        iota16 = lax.iota(jnp.int32, 16)

        def idx_copy(q, s):
            return pltpu.make_async_copy(
                idx_hbm.at[pl.ds(q * CW, CW)], idxb[s], isem[s])

        def gather_copy(s):
            return pltpu.make_async_copy(table_hbm.at[idxb[s]], rows[s], gsem[s])

        def start_gather(s):
            gather_copy(s).start()

        def write_copies(q, s):
            j = q // ic_per_j
            it = q % ic_per_j
            return [
                pltpu.make_async_copy(
                    tbuf[s].at[pl.ds(dt * DD, DD), pl.ds(0, CW)],
                    out_hbm.at[j, dt, it], wsem[s])
                for dt in range(DT)
            ]

        def transpose(s):
            for g in range(D // 16):
                did = g * 16 + iota16

                @pl.loop(0, CW, step=4)
                def _r(r):
                    for u in range(4):
                        v = rows[s][r + u, pl.ds(g * 16, 16)]
                        plsc.store_scatter(
                            tbuf[s], [did, iota16 * 0 + (r + u)], v)

        for b in range(DEPTH):
            idx_copy(q0 + b, b).start()
        for b in range(2):
            idx_copy(q0 + b, b).wait()
            start_gather(b)

        @pl.loop(0, q_per_w, step=DEPTH)
        def _group(t):
            for b in range(DEPTH):
                tq = t + b
                q = q0 + tq
                s = b
                gather_copy(s).wait()
                @pl.when(tq >= DEPTH)
                def _drainw():
                    for c in write_copies(q - DEPTH, s):
                        c.wait()
                @pl.when(tq + DEPTH < q_per_w)
                def _nexti():
                    idx_copy(q + DEPTH, s).start()
                transpose(s)
                for c in write_copies(q, s):
                    c.start()
                s2 = (b + 2) % DEPTH
                @pl.when(tq + 2 < q_per_w)
                def _nextg():
                    idx_copy(q + 2, s2).wait()
                    start_gather(s2)

        for b in range(DEPTH):
            tq = q_per_w - DEPTH + b
            for c in write_copies(q0 + tq, tq % DEPTH):
                c.wait()

    return k


@jax.jit
def kernel(sparse_table, indices):
    n0, n1 = indices.shape
    V, D = sparse_table.shape
    tt = sparse_table.T
    tail = jax.lax.slice(tt, (0, V - 128), (D, V))
    table2 = _make_relayout(V, D)(tt, tail)
    table_rm = table2.reshape(V, D)
    idx_t = indices.T.astype(jnp.int32).reshape(-1)
    out6 = _make_gather(V, D, n1, n0)(table_rm, idx_t)
    return out6.transpose(2, 4, 0, 1, 3).reshape(n0, n1, D)


# speedup vs baseline: 1.5561x; 1.3782x over previous
import functools

import jax
import jax.numpy as jnp
from jax import lax
from jax.experimental import pallas as pl
from jax.experimental.pallas import tpu as pltpu
from jax.experimental.pallas import tpu_sc as plsc

NC = 2
NS = 16
NW = NC * NS

CW = 128
DEPTH = 13
AHEAD = DEPTH - 2


def _make_gather(B, D, chunks_per_w):
    b_per_w = chunks_per_w * CW
    mesh = plsc.VectorSubcoreMesh(core_axis_name="c", subcore_axis_name="s")

    @functools.partial(
        pl.kernel,
        out_type=jax.ShapeDtypeStruct((B, D), jnp.float32),
        mesh=mesh,
        scratch_types=(
            [pltpu.VMEM((chunks_per_w, CW), jnp.int32)]
            + [pltpu.VMEM((CW, D), jnp.float32) for _ in range(DEPTH)]
            + [pltpu.SemaphoreType.DMA for _ in range(2 * DEPTH)]
        ),
        compiler_params=pltpu.CompilerParams(use_tc_tiling_on_sc=False),
    )
    def k(table_hbm, idx_hbm, out_hbm, idx_v, *bufs):
        rows = bufs[:DEPTH]
        gsem = bufs[DEPTH:2 * DEPTH]
        wsem = bufs[2 * DEPTH:]
        wid = lax.axis_index("s") * NC + lax.axis_index("c")
        base = wid * b_per_w
        pltpu.sync_copy(idx_hbm.at[wid], idx_v)

        def start_gather(q, s):
            pltpu.async_copy(table_hbm.at[idx_v.at[q]], rows[s], gsem[s])

        def wait_gather(q, s):
            pltpu.make_async_copy(table_hbm.at[idx_v.at[q]], rows[s], gsem[s]).wait()

        def start_write(q, s):
            pltpu.async_copy(rows[s], out_hbm.at[pl.ds(base + q * CW, CW)], wsem[s])

        def wait_write(q, s):
            pltpu.make_async_copy(
                rows[s], out_hbm.at[pl.ds(base + q * CW, CW)], wsem[s]).wait()

        for b in range(AHEAD):
            start_gather(b, b)

        @pl.loop(0, chunks_per_w, step=DEPTH)
        def _group(g):
            for b in range(DEPTH):
                q = g + b
                s = b
                wait_gather(q, s)
                start_write(q, s)
                sf = (b + AHEAD) % DEPTH

                @pl.when(q + AHEAD < chunks_per_w)
                def _refill():
                    @pl.when(q + AHEAD >= DEPTH)
                    def _drain():
                        wait_write(q + AHEAD - DEPTH, sf)
                    start_gather(q + AHEAD, sf)

        for i in range(DEPTH):
            q = chunks_per_w - DEPTH + i
            wait_write(q, q % DEPTH)

    return k


@jax.jit
def kernel(sparse_table, indices):
    n0, n1 = indices.shape
    D = sparse_table.shape[1]
    B = n0 * n1
    chunks_per_w = B // (NW * CW)
    idx = indices.reshape(NW, chunks_per_w, CW).astype(jnp.int32)
    out = _make_gather(B, D, chunks_per_w)(sparse_table, idx)
    return out.reshape(n0, n1, D)
